# baseline (speedup 1.0000x reference)
import functools

import jax
import jax.numpy as jnp
from jax import lax
from jax.experimental import pallas as pl
from jax.experimental.pallas import tpu as pltpu
from jax.experimental.pallas import tpu_sc as plsc

DIM_H = 1024
NV = 320
GR = 2
DCODE = 128
TB = 1024


def _argmax_body(hs_ref, w_ref, b_ref, idx0_ref, idx1_ref):
    hs = hs_ref[...]
    l = jnp.dot(hs, w_ref[...], preferred_element_type=jnp.float32)
    l = l + b_ref[...][None, :]
    iota = lax.broadcasted_iota(jnp.int32, (TB, GR * NV), 1)
    in_g1 = iota >= NV
    big = jnp.int32(2 ** 30)
    ninf = jnp.float32(float("-inf"))
    l0m = jnp.where(in_g1, ninf, l)
    l1m = jnp.where(in_g1, l, ninf)
    m0 = jnp.max(l0m, axis=1, keepdims=True)
    m1 = jnp.max(l1m, axis=1, keepdims=True)
    i0 = jnp.min(jnp.where(l0m == m0, iota, big), axis=1)
    i1 = jnp.min(jnp.where(l1m == m1, iota, big), axis=1)

    idx0_ref[...] = i0
    idx1_ref[...] = i1


def _argmax_call(hs2d, w_proj, b_proj):
    T = hs2d.shape[0]
    return pl.pallas_call(
        _argmax_body,
        grid=(T // TB,),
        in_specs=[
            pl.BlockSpec((TB, DIM_H), lambda i: (i, 0)),
            pl.BlockSpec((DIM_H, GR * NV), lambda i: (0, 0)),
            pl.BlockSpec((GR * NV,), lambda i: (0,)),
        ],
        out_specs=[
            pl.BlockSpec((TB,), lambda i: (i,)),
            pl.BlockSpec((TB,), lambda i: (i,)),
        ],
        out_shape=[
            jax.ShapeDtypeStruct((T,), jnp.int32),
            jax.ShapeDtypeStruct((T,), jnp.int32),
        ],
    )(hs2d, w_proj, b_proj)


def _dist_body(idx0_ref, idx1_ref, dist_ref):
    iota = lax.broadcasted_iota(jnp.int32, (NV, TB), 0)
    i0 = idx0_ref[...]
    i1 = idx1_ref[...] - NV
    dist_ref[0] = (iota == i0[None, :]).astype(jnp.float32)
    dist_ref[1] = (iota == i1[None, :]).astype(jnp.float32)


def _dist_call(idx0, idx1):
    T = idx0.shape[0]
    return pl.pallas_call(
        _dist_body,
        grid=(T // TB,),
        in_specs=[
            pl.BlockSpec((TB,), lambda i: (i,)),
            pl.BlockSpec((TB,), lambda i: (i,)),
        ],
        out_specs=pl.BlockSpec((GR, NV, TB), lambda i: (0, 0, i)),
        out_shape=jax.ShapeDtypeStruct((GR, NV, T), jnp.float32),
    )(idx0, idx1)


@functools.lru_cache(maxsize=None)
def _make_sc_gather(T):
    info = plsc.get_sparse_core_info()
    nw = info.num_cores * info.num_subcores
    t_per_w = T // nw
    mesh = plsc.VectorSubcoreMesh(core_axis_name="c", subcore_axis_name="s")

    @functools.partial(
        pl.kernel,
        mesh=mesh,
        out_type=jax.ShapeDtypeStruct((T, GR * DCODE), jnp.float32),
        scratch_types=[
            pltpu.VMEM((t_per_w,), jnp.int32),
            pltpu.VMEM((t_per_w,), jnp.int32),
            pltpu.VMEM((t_per_w, DCODE), jnp.float32),
            pltpu.VMEM((t_per_w, DCODE), jnp.float32),
            pltpu.SemaphoreType.DMA,
            pltpu.SemaphoreType.DMA,
        ],
    )
    def k(table_hbm, idx0_hbm, idx1_hbm, out_hbm, ia_v, ib_v, g0_v, g1_v,
          sem0, sem1):
        wid = lax.axis_index("s") * info.num_cores + lax.axis_index("c")
        base = wid * t_per_w
        pltpu.sync_copy(idx0_hbm.at[pl.ds(base, t_per_w)], ia_v)
        pltpu.sync_copy(idx1_hbm.at[pl.ds(base, t_per_w)], ib_v)
        c0 = pltpu.async_copy(table_hbm.at[ia_v], g0_v, sem0)
        c1 = pltpu.async_copy(table_hbm.at[ib_v], g1_v, sem1)
        c0.wait()
        c1.wait()
        pltpu.sync_copy(g0_v, out_hbm.at[pl.ds(base, t_per_w), pl.ds(0, DCODE)])
        pltpu.sync_copy(g1_v, out_hbm.at[pl.ds(base, t_per_w), pl.ds(DCODE, DCODE)])

    return k


def kernel(hidden_states, codevectors, w_proj, b_proj):
    B, S, H = hidden_states.shape
    T = B * S
    hs2d = hidden_states.reshape(T, H)
    idx0, idx1 = _argmax_call(hs2d, w_proj, b_proj)

    table = codevectors.reshape(GR * NV, DCODE)
    cv = _make_sc_gather(T)(table, idx0, idx1)
    cv = cv.reshape(B, S, GR * DCODE)
    dist_t = _dist_call(idx0, idx1)
    dist = jnp.transpose(dist_t, (2, 0, 1))
    return cv, dist
